# Initial kernel scaffold; baseline (speedup 1.0000x reference)
import jax
import jax.numpy as jnp
from jax.experimental import pallas as pl


def kernel(num_pos, pos_extra, v1, v2, batch_label, y, idx, memory_v1, memory_v2, all_sample_labels):
    raise NotImplementedError("write your pallas kernel here")


# speedup vs baseline: 9.9768x; 9.9768x over previous
import functools

import jax
import jax.numpy as jnp
from jax import lax
from jax.experimental import pallas as pl
from jax.experimental.pallas import tpu as pltpu
from jax.experimental.pallas import tpu_sc as plsc

_T = 0.07
_NC = 2
_NS = 16
_NW = _NC * _NS
_L = 16


def _make_sc_scores(B, K1, D):
    n_chunks = K1 // 128
    n_d = D // _L
    G = K1 // _L
    b_per_w = B // _NW

    mesh = plsc.VectorSubcoreMesh(
        core_axis_name="c", subcore_axis_name="s",
        num_cores=_NC, num_subcores=_NS)

    @functools.partial(
        pl.kernel,
        out_type=[jax.ShapeDtypeStruct((B, K1), jnp.float32),
                  jax.ShapeDtypeStruct((B, K1), jnp.float32)],
        mesh=mesh,
        compiler_params=pltpu.CompilerParams(
            needs_layout_passes=False, use_tc_tiling_on_sc=False),
        scratch_types=[
            pltpu.VMEM((n_chunks, 128), jnp.int32),
            pltpu.VMEM((K1, D), jnp.float32),
            pltpu.VMEM((K1, D), jnp.float32),
            pltpu.VMEM((D,), jnp.float32),
            pltpu.VMEM((D,), jnp.float32),
            pltpu.VMEM((K1,), jnp.float32),
            pltpu.VMEM((K1,), jnp.float32),
            pltpu.SemaphoreType.DMA,
        ],
    )
    def sc_scores(mem1, mem2, v1h, v2h, idxh, s1h, s2h,
                  idx_v, rows1_v, rows2_v, v1_v, v2_v, s1_v, s2_v, gsem):
        wid = lax.axis_index("s") * _NC + lax.axis_index("c")
        iota = lax.iota(jnp.int32, _L)
        last = jnp.full((_L,), _L - 1, jnp.int32)

        def b_body(i, carry):
            b = wid * b_per_w + i
            pltpu.sync_copy(idxh.at[b], idx_v)
            pltpu.sync_copy(v1h.at[b], v1_v)
            pltpu.sync_copy(v2h.at[b], v2_v)
            cps = []
            for j in range(n_chunks):
                dst = pl.ds(j * 128, 128)
                cps.append(pltpu.async_copy(
                    mem1.at[idx_v.at[j]], rows1_v.at[dst], gsem))
                cps.append(pltpu.async_copy(
                    mem2.at[idx_v.at[j]], rows2_v.at[dst], gsem))
            for cp in cps:
                cp.wait()
            vc1 = [v1_v[pl.ds(c * _L, _L)] for c in range(n_d)]
            vc2 = [v2_v[pl.ds(c * _L, _L)] for c in range(n_d)]

            def g_body(g, c2):
                sel1 = jnp.zeros((_L,), jnp.float32)
                sel2 = jnp.zeros((_L,), jnp.float32)
                for kk in range(_L):
                    k = g * _L + kk
                    p1 = rows2_v[k, pl.ds(0, _L)] * vc1[0]
                    p2 = rows1_v[k, pl.ds(0, _L)] * vc2[0]
                    for c in range(1, n_d):
                        p1 = p1 + rows2_v[k, pl.ds(c * _L, _L)] * vc1[c]
                        p2 = p2 + rows1_v[k, pl.ds(c * _L, _L)] * vc2[c]
                    t1 = plsc.cumsum(p1).at[last].get(mode="promise_in_bounds")
                    t2 = plsc.cumsum(p2).at[last].get(mode="promise_in_bounds")
                    m = iota == kk
                    sel1 = jnp.where(m, t1, sel1)
                    sel2 = jnp.where(m, t2, sel2)
                krows = g * _L + iota
                plsc.store_scatter(s1_v, [krows], sel1)
                plsc.store_scatter(s2_v, [krows], sel2)
                return c2
            lax.fori_loop(0, G, g_body, 0)
            pltpu.sync_copy(s1_v, s1h.at[b])
            pltpu.sync_copy(s2_v, s2h.at[b])
            return carry

        lax.fori_loop(0, b_per_w, b_body, 0)

    return sc_scores


def _make_norm(B, K1, out_size):
    def body(s1_ref, s2_ref, o1_ref, o2_ref):
        e1 = jnp.exp(s1_ref[...] * (1.0 / _T))
        e2 = jnp.exp(s2_ref[...] * (1.0 / _T))
        z1 = jnp.mean(e1) * out_size
        z2 = jnp.mean(e2) * out_size
        o1_ref[...] = e1 / z1
        o2_ref[...] = e2 / z2

    return pl.pallas_call(
        body,
        out_shape=[jax.ShapeDtypeStruct((B, K1), jnp.float32),
                   jax.ShapeDtypeStruct((B, K1), jnp.float32)],
    )


def kernel(num_pos, pos_extra, v1, v2, batch_label, y, idx, memory_v1,
           memory_v2, all_sample_labels):
    B, D = v1.shape
    K1 = idx.shape[1]
    out_size = memory_v1.shape[0]
    idx3 = idx.astype(jnp.int32).reshape(B, K1 // 128, 128)
    mem2_r = memory_v2.astype(jnp.bfloat16).astype(jnp.float32)
    v1_r = v1.astype(jnp.bfloat16).astype(jnp.float32)
    sc_scores = _make_sc_scores(B, K1, D)
    s1, s2 = sc_scores(memory_v1, mem2_r, v1_r, v2, idx3)
    out_v1, out_v2 = _make_norm(B, K1, out_size)(s1, s2)
    return (out_v1[:, :, None], out_v2[:, :, None])


# speedup vs baseline: 14.3969x; 1.4430x over previous
import functools

import jax
import jax.numpy as jnp
from jax import lax
from jax.experimental import pallas as pl
from jax.experimental.pallas import tpu as pltpu
from jax.experimental.pallas import tpu_sc as plsc

_T = 0.07
_NC = 2
_NS = 16
_NW = _NC * _NS
_L = 16


def _make_sc_scores(B, K1, D):
    n_chunks = K1 // 128
    n_d = D // _L
    G = K1 // _L
    b_per_w = B // _NW

    mesh = plsc.VectorSubcoreMesh(
        core_axis_name="c", subcore_axis_name="s",
        num_cores=_NC, num_subcores=_NS)

    @functools.partial(
        pl.kernel,
        out_type=[jax.ShapeDtypeStruct((B, K1), jnp.float32),
                  jax.ShapeDtypeStruct((B, K1), jnp.float32)],
        mesh=mesh,
        compiler_params=pltpu.CompilerParams(
            needs_layout_passes=False, use_tc_tiling_on_sc=False),
        scratch_types=[
            pltpu.VMEM((b_per_w, n_chunks, 128), jnp.int32),
            pltpu.VMEM((b_per_w, D), jnp.float32),
            pltpu.VMEM((b_per_w, D), jnp.float32),
            pltpu.VMEM((K1, D), jnp.float32),
            pltpu.VMEM((K1, D), jnp.float32),
            pltpu.VMEM((2 * K1,), jnp.float32),
            pltpu.VMEM((2 * K1,), jnp.float32),
            pltpu.SemaphoreType.DMA,
            pltpu.SemaphoreType.DMA,
            pltpu.SemaphoreType.DMA,
        ],
    )
    def sc_scores(mem1, mem2, v1h, v2h, idxh, s1h, s2h,
                  idx_all, v1_all, v2_all, rows1_v, rows2_v, s1_v, s2_v,
                  sem1, sem2, semo):
        wid = lax.axis_index("s") * _NC + lax.axis_index("c")
        b0 = wid * b_per_w
        iota = lax.iota(jnp.int32, _L)
        last = jnp.full((_L,), _L - 1, jnp.int32)

        pltpu.sync_copy(idxh.at[pl.ds(b0, b_per_w)], idx_all)
        pltpu.sync_copy(v1h.at[pl.ds(b0, b_per_w)], v1_all)
        pltpu.sync_copy(v2h.at[pl.ds(b0, b_per_w)], v2_all)

        def issue(mem, rows_v, sem, i):
            for j in range(n_chunks):
                pltpu.async_copy(
                    mem.at[idx_all.at[i, j]],
                    rows_v.at[pl.ds(j * 128, 128)], sem)

        def wait_bank(mem, rows_v, sem):
            pltpu.make_async_copy(mem.at[pl.ds(0, K1)], rows_v, sem).wait()

        def compute_bank(rows_v, vcs, s_v, buf):
            def g_body(g, c2):
                sel = jnp.zeros((_L,), jnp.float32)
                for kk in range(_L):
                    k = g * _L + kk
                    p = rows_v[k, pl.ds(0, _L)] * vcs[0]
                    for c in range(1, n_d):
                        p = p + rows_v[k, pl.ds(c * _L, _L)] * vcs[c]
                    t = plsc.cumsum(p).at[last].get(mode="promise_in_bounds")
                    sel = jnp.where(iota == kk, t, sel)
                plsc.store_scatter(s_v, [buf * K1 + g * _L + iota], sel)
                return c2
            lax.fori_loop(0, G, g_body, 0)

        issue(mem1, rows1_v, sem1, 0)
        issue(mem2, rows2_v, sem2, 0)

        def b_body(i, carry):
            b = b0 + i
            buf = lax.rem(i, 2)

            bslot = pl.ds(buf * K1, K1)

            @pl.when(i >= 2)
            def _():
                pltpu.make_async_copy(s1h.at[b], s1_v.at[bslot], semo).wait()
                pltpu.make_async_copy(s2h.at[b], s2_v.at[bslot], semo).wait()

            wait_bank(mem1, rows1_v, sem1)
            vcs2 = [v2_all[i, pl.ds(c * _L, _L)] for c in range(n_d)]
            compute_bank(rows1_v, vcs2, s2_v, buf)

            @pl.when(i + 1 < b_per_w)
            def _():
                issue(mem1, rows1_v, sem1, i + 1)

            wait_bank(mem2, rows2_v, sem2)
            vcs1 = [v1_all[i, pl.ds(c * _L, _L)] for c in range(n_d)]
            compute_bank(rows2_v, vcs1, s1_v, buf)

            @pl.when(i + 1 < b_per_w)
            def _():
                issue(mem2, rows2_v, sem2, i + 1)

            pltpu.async_copy(s1_v.at[bslot], s1h.at[b], semo)
            pltpu.async_copy(s2_v.at[bslot], s2h.at[b], semo)
            return carry

        lax.fori_loop(0, b_per_w, b_body, 0)

        for i in (b_per_w - 2, b_per_w - 1):
            slot = pl.ds((i % 2) * K1, K1)
            pltpu.make_async_copy(
                s1h.at[b0 + i], s1_v.at[slot], semo).wait()
            pltpu.make_async_copy(
                s2h.at[b0 + i], s2_v.at[slot], semo).wait()

    return sc_scores


def _make_norm(B, K1, out_size):
    def body(s1_ref, s2_ref, o1_ref, o2_ref):
        e1 = jnp.exp(s1_ref[...] * (1.0 / _T))
        e2 = jnp.exp(s2_ref[...] * (1.0 / _T))
        z1 = jnp.mean(e1) * out_size
        z2 = jnp.mean(e2) * out_size
        o1_ref[...] = e1 / z1
        o2_ref[...] = e2 / z2

    return pl.pallas_call(
        body,
        out_shape=[jax.ShapeDtypeStruct((B, K1), jnp.float32),
                   jax.ShapeDtypeStruct((B, K1), jnp.float32)],
    )


def kernel(num_pos, pos_extra, v1, v2, batch_label, y, idx, memory_v1,
           memory_v2, all_sample_labels):
    B, D = v1.shape
    K1 = idx.shape[1]
    out_size = memory_v1.shape[0]
    idx3 = idx.astype(jnp.int32).reshape(B, K1 // 128, 128)
    mem2_r = memory_v2.astype(jnp.bfloat16).astype(jnp.float32)
    v1_r = v1.astype(jnp.bfloat16).astype(jnp.float32)
    sc_scores = _make_sc_scores(B, K1, D)
    s1, s2 = sc_scores(memory_v1, mem2_r, v1_r, v2, idx3)
    out_v1, out_v2 = _make_norm(B, K1, out_size)(s1, s2)
    return (out_v1[:, :, None], out_v2[:, :, None])
